# baseline (speedup 1.0000x reference)
import jax
import jax.numpy as jnp
from jax.experimental import pallas as pl

_HEADS = 8
_CONCAT = [True, True, False]
_NUM_LAYERS = 3
_DIMS_OUT = [8, 8, 8]


def _copy_kernel(x_ref, o_ref):
    o_ref[...] = x_ref[...]


def _gatv2_conv(x, edge_index, edge_attr, p, concat, H, C):
    n = x.shape[0]
    src = edge_index[0]
    dst = edge_index[1]
    loop = jnp.arange(n, dtype=src.dtype)
    src_f = jnp.concatenate([src, loop])
    dst_f = jnp.concatenate([dst, loop])
    fill = jnp.broadcast_to(edge_attr.mean(axis=0, keepdims=True), (n, edge_attr.shape[1]))
    ea = jnp.concatenate([edge_attr, fill], axis=0)
    xl = (x @ p['Wl'] + p['bl']).reshape(n, H, C)
    xr = (x @ p['Wr'] + p['br']).reshape(n, H, C)
    e = (ea @ p['We']).reshape(-1, H, C)
    m = xl[src_f] + xr[dst_f] + e
    m = jax.nn.leaky_relu(m, 0.2)
    alpha = (m * p['att']).sum(-1)
    amax = jax.ops.segment_max(alpha, dst_f, num_segments=n)
    alpha = jnp.exp(alpha - amax[dst_f])
    denom = jax.ops.segment_sum(alpha, dst_f, num_segments=n)
    alpha = alpha / (denom[dst_f] + 1e-16)
    msg = xl[src_f] * alpha[:, :, None]
    out = jax.ops.segment_sum(msg, dst_f, num_segments=n)
    if concat:
        out = out.reshape(n, H * C)
    else:
        out = out.mean(axis=1)
    return out + p['bias']


def _head(x, p):
    h = jax.nn.relu(x @ p['W1'] + p['b1'])
    return h @ p['W2'] + p['b2']


def kernel(x, edge_index, edge_attr, conv_params, node_head_params, edge_head_params):
    h = x
    for i in range(_NUM_LAYERS):
        h = _gatv2_conv(h, edge_index, edge_attr, conv_params[i], _CONCAT[i], _HEADS, _DIMS_OUT[i])
        h = jax.nn.relu(h)
    h = pl.pallas_call(
        _copy_kernel,
        out_shape=jax.ShapeDtypeStruct(h.shape, h.dtype),
    )(h)
    node_out = jax.nn.log_softmax(_head(h, node_head_params), axis=1)
    edges = h[edge_index[0]] + h[edge_index[1]]
    edge_out = jax.nn.log_softmax(_head(edges, edge_head_params), axis=1)
    return (node_out, edge_out)


# speedup vs baseline: 22.4911x; 22.4911x over previous
import functools

import jax
import jax.numpy as jnp
from jax import lax
from jax.experimental import pallas as pl
from jax.experimental.pallas import tpu as pltpu
from jax.experimental.pallas import tpu_sc as plsc

_H = 8
_C = 8
_HC = _H * _C
_NC = 2
_NS = 16
_NW = _NC * _NS
_EB = 128

_f32 = jnp.float32
_i32 = jnp.int32


def _pick(n, prefs):
    for p in prefs:
        if n % p == 0:
            return p
    return 1


def _ea_stats_body(ea_ref, mean_ref, mn_ref, mx_ref, *, e_real):
    blk = ea_ref[...]
    mean_ref[...] = jnp.full((1, 1), jnp.sum(blk) / e_real, _f32)
    mn_ref[...] = jnp.full((1, 1), jnp.min(blk), _f32)
    mx_ref[...] = jnp.full((1, 1), jnp.max(blk), _f32)


def _ea_stats(ea):
    e_real = ea.shape[0]
    rows = e_real // 128
    ea2 = ea.reshape(rows, 128)
    out = [jax.ShapeDtypeStruct((1, 1), _f32)] * 3
    return pl.pallas_call(
        functools.partial(_ea_stats_body, e_real=float(e_real)),
        out_shape=out,
    )(ea2)


def _prep_body(h_ref, wl_ref, bl_ref, wr_ref, br_ref, we_ref, att_ref,
               eamean_ref, eamin_ref, eamax_ref,
               xl_ref, xr_ref, self8_ref, usc_ref, utc_ref,
               mnl_ref, mxl_ref, mnr_ref, mxr_ref, *, bn):
    i = pl.program_id(0)
    hb = h_ref[...]
    xlb = jnp.dot(hb, wl_ref[...], preferred_element_type=_f32) + bl_ref[...]
    xrb = jnp.dot(hb, wr_ref[...], preferred_element_type=_f32) + br_ref[...]
    xl_ref[...] = xlb
    xr_ref[...] = xrb
    we = we_ref[...]
    att = att_ref[...]
    mean = eamean_ref[0, 0]
    m = xlb + xrb + mean * we
    m = jnp.maximum(m, 0.2 * m)
    t = (m * att).reshape(bn, _H, _C)
    self8_ref[...] = jnp.sum(t, axis=-1)

    bmnl = jnp.min(xlb, axis=0, keepdims=True)
    bmxl = jnp.max(xlb, axis=0, keepdims=True)
    bmnr = jnp.min(xrb, axis=0, keepdims=True)
    bmxr = jnp.max(xrb, axis=0, keepdims=True)

    @pl.when(i == 0)
    def _():
        mnl_ref[...] = bmnl
        mxl_ref[...] = bmxl
        mnr_ref[...] = bmnr
        mxr_ref[...] = bmxr

    @pl.when(i > 0)
    def _():
        mnl_ref[...] = jnp.minimum(mnl_ref[...], bmnl)
        mxl_ref[...] = jnp.maximum(mxl_ref[...], bmxl)
        mnr_ref[...] = jnp.minimum(mnr_ref[...], bmnr)
        mxr_ref[...] = jnp.maximum(mxr_ref[...], bmxr)

    @pl.when(i == pl.num_programs(0) - 1)
    def _():
        ea_lo = eamin_ref[0, 0]
        ea_hi = eamax_ref[0, 0]
        e_hi = jnp.maximum(we * ea_lo, we * ea_hi)
        e_lo = jnp.minimum(we * ea_lo, we * ea_hi)
        bhi = mxl_ref[...] + mxr_ref[...] + e_hi
        blo = mnl_ref[...] + mnr_ref[...] + e_lo
        mhi = jnp.maximum(bhi, 0.2 * bhi)
        mlo = jnp.maximum(blo, 0.2 * blo)
        chi = jnp.maximum(att * mhi, att * mlo)
        u8 = jnp.sum(chi.reshape(1, _H, _C), axis=-1)
        utc_ref[...] = u8
        usc_ref[...] = jnp.broadcast_to(u8.reshape(_H, 1), (_H, 16))


def _prep(h, p, stats):
    n, din = h.shape
    bn = _pick(n, (2000, 1000, 400, 200, 100, 50, 25, 16, 8, 4, 2, 1))
    g = n // bn
    full = lambda a: pl.BlockSpec(a.shape, lambda i: tuple([0] * a.ndim))
    wl = p['Wl']
    bl = p['bl'].reshape(1, _HC)
    wr = p['Wr']
    br = p['br'].reshape(1, _HC)
    we = p['We'].reshape(1, _HC)
    att = p['att'].reshape(1, _HC)
    eamean, eamin, eamax = stats
    outs = [
        jax.ShapeDtypeStruct((n, _HC), _f32),
        jax.ShapeDtypeStruct((n, _HC), _f32),
        jax.ShapeDtypeStruct((n, _H), _f32),
        jax.ShapeDtypeStruct((_H, 16), _f32),
        jax.ShapeDtypeStruct((1, _H), _f32),
    ]
    return pl.pallas_call(
        functools.partial(_prep_body, bn=bn),
        grid=(g,),
        in_specs=[
            pl.BlockSpec((bn, din), lambda i: (i, 0)),
            full(wl), full(bl), full(wr), full(br), full(we), full(att),
            full(eamean), full(eamin), full(eamax),
        ],
        out_specs=[
            pl.BlockSpec((bn, _HC), lambda i: (i, 0)),
            pl.BlockSpec((bn, _HC), lambda i: (i, 0)),
            pl.BlockSpec((bn, _H), lambda i: (i, 0)),
            pl.BlockSpec((_H, 16), lambda i: (0, 0)),
            pl.BlockSpec((1, _H), lambda i: (0, 0)),
        ],
        out_shape=outs,
        scratch_shapes=[pltpu.VMEM((1, _HC), _f32)] * 4,
    )(h, wl, bl, wr, br, we, att, eamean, eamin, eamax)


def _combine_body(acc_ref, den_ref, xl_ref, self8_ref, utc_ref, bias_ref,
                  out_ref, *, bn, concat):
    accb = acc_ref[...]
    denb = den_ref[...]
    acc64 = jnp.concatenate([accb[0], accb[1]], axis=1)
    den8 = jnp.concatenate([denb[0][:, :4], denb[1][:, :4]], axis=1)
    ws = jnp.exp(self8_ref[...] - utc_ref[...])
    xlb = xl_ref[...]
    wsr = jnp.broadcast_to(ws.reshape(bn, _H, 1), (bn, _H, _C)).reshape(bn, _HC)
    num = acc64 + wsr * xlb
    d = den8 + ws + 1e-16
    dr = jnp.broadcast_to(d.reshape(bn, _H, 1), (bn, _H, _C)).reshape(bn, _HC)
    o = num / dr
    if concat:
        out_ref[...] = jnp.maximum(o + bias_ref[...], 0.0)
    else:
        s = o[:, 0:_C]
        for hh in range(1, _H):
            s = s + o[:, hh * _C:(hh + 1) * _C]
        o8 = s * (1.0 / _H) + bias_ref[...]
        o8 = jnp.maximum(o8, 0.0)
        out_ref[...] = jnp.concatenate([o8, jnp.zeros((bn, 8), _f32)], axis=1)


def _combine(acc, den, xl, self8, utc, bias, concat):
    n = xl.shape[0]
    bn = _pick(n, (2000, 1000, 400, 200, 100, 50, 25, 16, 8, 4, 2, 1))
    g = n // bn
    bias2 = bias.reshape(1, -1)
    odim = _HC if concat else 16
    return pl.pallas_call(
        functools.partial(_combine_body, bn=bn, concat=concat),
        grid=(g,),
        in_specs=[
            pl.BlockSpec((2, bn, 32), lambda i: (0, i, 0)),
            pl.BlockSpec((2, bn, 8), lambda i: (0, i, 0)),
            pl.BlockSpec((bn, _HC), lambda i: (i, 0)),
            pl.BlockSpec((bn, _H), lambda i: (i, 0)),
            pl.BlockSpec((1, _H), lambda i: (0, 0)),
            pl.BlockSpec(bias2.shape, lambda i: (0, 0)),
        ],
        out_specs=pl.BlockSpec((bn, odim), lambda i: (i, 0)),
        out_shape=jax.ShapeDtypeStruct((n, odim), _f32),
    )(acc, den, xl, self8, utc, bias2)


def _head_body(h_ref, w1_ref, b1_ref, w2_ref, b2_ref, out_ref):
    hb = h_ref[...]
    z = jnp.dot(hb, w1_ref[...], preferred_element_type=_f32) + b1_ref[...]
    z = jnp.maximum(z, 0.0)
    z2 = jnp.dot(z, w2_ref[...], preferred_element_type=_f32) + b2_ref[...]
    m = jnp.max(z2, axis=1, keepdims=True)
    zz = z2 - m
    lse = jnp.log(jnp.sum(jnp.exp(zz), axis=1, keepdims=True))
    out_ref[...] = zz - lse


def _mlp_head(feat, p):
    n = feat.shape[0]
    bn = _pick(n, (2000, 1000, 400, 200, 100, 50, 25, 16, 8, 4, 2, 1))
    g = n // bn
    w1 = jnp.concatenate([p['W1'], jnp.zeros((16 - p['W1'].shape[0], p['W1'].shape[1]), _f32)], axis=0)
    b1 = p['b1'].reshape(1, -1)
    w2 = p['W2']
    b2 = p['b2'].reshape(1, -1)
    full = lambda a: pl.BlockSpec(a.shape, lambda i: tuple([0] * a.ndim))
    return pl.pallas_call(
        _head_body,
        grid=(g,),
        in_specs=[
            pl.BlockSpec((bn, 16), lambda i: (i, 0)),
            full(w1), full(b1), full(w2), full(b2),
        ],
        out_specs=pl.BlockSpec((bn, 2), lambda i: (i, 0)),
        out_shape=jax.ShapeDtypeStruct((n, 2), _f32),
    )(feat, w1, b1, w2, b2)


def _mesh():
    return plsc.VectorSubcoreMesh(
        core_axis_name="c", subcore_axis_name="s",
        num_cores=_NC, num_subcores=_NS)


_SC_PARAMS = pltpu.CompilerParams(
    needs_layout_passes=False, use_tc_tiling_on_sc=False)


def _passA_body(xl_hbm, xr_hbm, ei_hbm, ea_hbm, web_hbm, attb_hbm, ub_hbm,
                w_hbm, webuf, attbuf, ubuf, eib, eab, xlb, xrb, wb, sem,
                *, e_real, nchunk):
    c = lax.axis_index("c")
    s = lax.axis_index("s")
    t = c * _NS + s
    pltpu.sync_copy(web_hbm, webuf)
    pltpu.sync_copy(attb_hbm, attbuf)
    pltpu.sync_copy(ub_hbm, ubuf)
    tile_base = t * (nchunk * _EB)

    def chunk(i, carry):
        base = tile_base + i * _EB
        pltpu.sync_copy(ei_hbm.at[:, pl.ds(base, _EB)], eib)
        pltpu.sync_copy(ea_hbm.at[pl.ds(base, _EB)], eab)
        pltpu.async_copy(xl_hbm.at[eib.at[0]], xlb, sem).wait()
        pltpu.async_copy(xr_hbm.at[eib.at[1]], xrb, sem).wait()

        def block(eb, carry2):
            ids = lax.iota(_i32, 16) + eb * 16
            eav = eab[pl.ds(eb * 16, 16)]
            logits = [jnp.zeros((16,), _f32)] * _H
            for j in range(_HC):
                jv = jnp.full((16,), j, _i32)
                xc = plsc.load_gather(xlb, [ids, jv])
                rc = plsc.load_gather(xrb, [ids, jv])
                mm = xc + rc + eav * webuf[j]
                mm = jnp.maximum(mm, 0.2 * mm)
                logits[j // _C] = logits[j // _C] + mm * attbuf[j]
            gid = base + eb * 16 + lax.iota(_i32, 16)
            msk = gid < e_real
            for h in range(_H):
                wv = jnp.exp(logits[h] - ubuf[h])
                wv = jnp.where(msk, wv, 0.0)
                wb[h, pl.ds(eb * 16, 16)] = wv
            return carry2

        lax.fori_loop(0, _EB // 16, block, 0)
        pltpu.sync_copy(wb, w_hbm.at[:, pl.ds(base, _EB)])
        return carry

    lax.fori_loop(0, nchunk, chunk, 0)


def _passA(xl, xr, ei_pad, ea_pad, web, attb, usc, e_real, epad):
    nchunk = epad // (_NW * _EB)
    f = pl.kernel(
        functools.partial(_passA_body, e_real=e_real, nchunk=nchunk),
        out_type=[jax.ShapeDtypeStruct((_H, epad), _f32)],
        mesh=_mesh(),
        compiler_params=_SC_PARAMS,
        scratch_types=[
            pltpu.VMEM((_HC, 16), _f32),
            pltpu.VMEM((_HC, 16), _f32),
            pltpu.VMEM((_H, 16), _f32),
            pltpu.VMEM((2, _EB), _i32),
            pltpu.VMEM((_EB,), _f32),
            pltpu.VMEM((_EB, _HC), _f32),
            pltpu.VMEM((_EB, _HC), _f32),
            pltpu.VMEM((_H, _EB), _f32),
            pltpu.SemaphoreType.DMA,
        ],
    )
    return f(xl, xr, ei_pad, ea_pad, web, attb, usc)[0]


def _passB_body(xl0_hbm, xl1_hbm, ei_hbm, w_hbm, z32_hbm,
                accout, eib, xlc, wbv, msg, sem, acc_sh, *, rpt, nchunk):
    c = lax.axis_index("c")
    s = lax.axis_index("s")
    r0 = s * rpt
    pltpu.sync_copy(z32_hbm.at[pl.ds(r0, rpt)], acc_sh.at[pl.ds(r0, rpt)])
    plsc.subcore_barrier()
    stripe = nchunk * _EB

    def run(core, xsrc_hbm):
        def chunk(i, carry):
            base = s * stripe + i * _EB
            pltpu.sync_copy(ei_hbm.at[:, pl.ds(base, _EB)], eib)
            pltpu.async_copy(xsrc_hbm.at[eib.at[0]], xlc, sem).wait()
            pltpu.sync_copy(
                w_hbm.at[pl.ds(core * 4, 4), pl.ds(base, _EB)], wbv)

            def block(eb, carry2):
                ids = lax.iota(_i32, 16) + eb * 16
                wv = [plsc.load_gather(wbv, [jnp.full((16,), h, _i32), ids])
                      for h in range(4)]
                for j in range(32):
                    jv = jnp.full((16,), j, _i32)
                    col = plsc.load_gather(xlc, [ids, jv])
                    plsc.store_scatter(msg, [ids, jv], col * wv[j // _C])
                return carry2

            lax.fori_loop(0, _EB // 16, block, 0)
            pltpu.sync_copy(msg, acc_sh.at[eib.at[1]], add=True)
            return carry

        lax.fori_loop(0, nchunk, chunk, 0)
        plsc.subcore_barrier()
        pltpu.sync_copy(acc_sh.at[pl.ds(r0, rpt)],
                        accout.at[core, pl.ds(r0, rpt)])

    @pl.when(c == 0)
    def _():
        run(0, xl0_hbm)

    @pl.when(c == 1)
    def _():
        run(1, xl1_hbm)


def _passB(xl0, xl1, ei_pad, w, n, epad):
    rpt = n // _NS
    nchunk = epad // (_NS * _EB)
    z32 = jnp.zeros((n, 32), _f32)
    f = pl.kernel(
        functools.partial(_passB_body, rpt=rpt, nchunk=nchunk),
        out_type=[jax.ShapeDtypeStruct((_NC, n, 32), _f32)],
        mesh=_mesh(),
        compiler_params=_SC_PARAMS,
        scratch_types=[
            pltpu.VMEM((2, _EB), _i32),
            pltpu.VMEM((_EB, 32), _f32),
            pltpu.VMEM((4, _EB), _f32),
            pltpu.VMEM((_EB, 32), _f32),
            pltpu.SemaphoreType.DMA,
            pltpu.VMEM_SHARED((n, 32), _f32),
        ],
    )
    return f(xl0, xl1, ei_pad, w, z32)[0]


def _passBd_body(ei_hbm, w_hbm, z8_hbm, z8row_hbm, denout,
                 dstb, wbv, denb, acc_den_sh, *, rpt, nchunk):
    c = lax.axis_index("c")
    s = lax.axis_index("s")
    r0 = s * rpt
    pltpu.sync_copy(z8_hbm.at[pl.ds(r0, rpt)], acc_den_sh.at[pl.ds(r0, rpt)])
    pltpu.sync_copy(z8row_hbm, denb)
    plsc.subcore_barrier()
    stripe = nchunk * _EB

    def run(core):
        def chunk(i, carry):
            base = s * stripe + i * _EB
            pltpu.sync_copy(ei_hbm.at[:, pl.ds(base, _EB)], dstb)
            pltpu.sync_copy(
                w_hbm.at[pl.ds(core * 4, 4), pl.ds(base, _EB)], wbv)

            def block(eb, carry2):
                ids = lax.iota(_i32, 16) + eb * 16
                for h in range(4):
                    wv = plsc.load_gather(wbv, [jnp.full((16,), h, _i32), ids])
                    plsc.store_scatter(denb, [ids, jnp.full((16,), h, _i32)],
                                       wv)
                return carry2

            lax.fori_loop(0, _EB // 16, block, 0)
            pltpu.sync_copy(denb, acc_den_sh.at[dstb.at[1]], add=True)
            return carry

        lax.fori_loop(0, nchunk, chunk, 0)
        plsc.subcore_barrier()
        pltpu.sync_copy(acc_den_sh.at[pl.ds(r0, rpt)],
                        denout.at[core, pl.ds(r0, rpt)])

    @pl.when(c == 0)
    def _():
        run(0)

    @pl.when(c == 1)
    def _():
        run(1)


def _passBd(ei_pad, w, n, epad):
    rpt = n // _NS
    nchunk = epad // (_NS * _EB)
    z8 = jnp.zeros((n, 8), _f32)
    z8row = jnp.zeros((_EB, 8), _f32)
    f = pl.kernel(
        functools.partial(_passBd_body, rpt=rpt, nchunk=nchunk),
        out_type=[jax.ShapeDtypeStruct((_NC, n, 8), _f32)],
        mesh=_mesh(),
        compiler_params=_SC_PARAMS,
        scratch_types=[
            pltpu.VMEM((2, _EB), _i32),
            pltpu.VMEM((4, _EB), _f32),
            pltpu.VMEM((_EB, 8), _f32),
            pltpu.VMEM_SHARED((n, 8), _f32),
        ],
    )
    return f(ei_pad, w, z8, z8row)[0]


def _passC_body(h16_hbm, ei_hbm, efeat_hbm, eib, g1, g2, efb, sem, *, nchunk):
    c = lax.axis_index("c")
    s = lax.axis_index("s")
    t = c * _NS + s
    tile_base = t * (nchunk * _EB)

    def chunk(i, carry):
        base = tile_base + i * _EB
        pltpu.sync_copy(ei_hbm.at[:, pl.ds(base, _EB)], eib)
        pltpu.async_copy(h16_hbm.at[eib.at[0]], g1, sem).wait()
        pltpu.async_copy(h16_hbm.at[eib.at[1]], g2, sem).wait()

        def row(r, carry2):
            efb[r] = g1[r] + g2[r]
            return carry2

        lax.fori_loop(0, _EB, row, 0)
        pltpu.sync_copy(efb, efeat_hbm.at[pl.ds(base, _EB)])
        return carry

    lax.fori_loop(0, nchunk, chunk, 0)


def _passC(h16, ei_pad, epad):
    nchunk = epad // (_NW * _EB)
    f = pl.kernel(
        functools.partial(_passC_body, nchunk=nchunk),
        out_type=[jax.ShapeDtypeStruct((epad, 16), _f32)],
        mesh=_mesh(),
        compiler_params=_SC_PARAMS,
        scratch_types=[
            pltpu.VMEM((2, _EB), _i32),
            pltpu.VMEM((_EB, 16), _f32),
            pltpu.VMEM((_EB, 16), _f32),
            pltpu.VMEM((_EB, 16), _f32),
            pltpu.SemaphoreType.DMA,
        ],
    )
    return f(h16, ei_pad)[0]


def kernel(x, edge_index, edge_attr, conv_params, node_head_params,
           edge_head_params):
    n = x.shape[0]
    e = edge_index.shape[1]
    unit = _NW * _EB
    epad = ((e + unit - 1) // unit) * unit
    pad = epad - e
    ei_pad = jnp.concatenate(
        [edge_index, jnp.zeros((2, pad), edge_index.dtype)], axis=1)
    ea_flat = edge_attr[:, 0]
    ea_pad = jnp.concatenate([ea_flat, jnp.zeros((pad,), _f32)])

    stats = _ea_stats(ea_flat)

    concat_flags = [True, True, False]
    h = x
    for li in range(3):
        p = conv_params[li]
        xl, xr, self8, usc, utc = _prep(h, p, stats)
        web = jnp.broadcast_to(p['We'].reshape(_HC, 1), (_HC, 16))
        attb = jnp.broadcast_to(p['att'].reshape(_HC, 1), (_HC, 16))
        w = _passA(xl, xr, ei_pad, ea_pad, web, attb, usc, e, epad)
        acc = _passB(xl[:, :32], xl[:, 32:], ei_pad, w, n, epad)
        den = _passBd(ei_pad, w, n, epad)
        h = _combine(acc, den, xl, self8, utc, p['bias'], concat_flags[li])

    node_out = _mlp_head(h, node_head_params)
    efeat = _passC(h, ei_pad, epad)[:e]
    edge_out = _mlp_head(efeat, edge_head_params)
    return (node_out, edge_out)


# speedup vs baseline: 23.2188x; 1.0324x over previous
import functools

import jax
import jax.numpy as jnp
from jax import lax
from jax.experimental import pallas as pl
from jax.experimental.pallas import tpu as pltpu
from jax.experimental.pallas import tpu_sc as plsc

_H = 8
_C = 8
_HC = _H * _C
_NC = 2
_NS = 16
_NW = _NC * _NS
_EB = 128

_f32 = jnp.float32
_i32 = jnp.int32


def _pick(n, prefs):
    for p in prefs:
        if n % p == 0:
            return p
    return 1


def _ea_stats_body(ea_ref, mean_ref, mn_ref, mx_ref, *, e_real):
    blk = ea_ref[...]
    mean_ref[...] = jnp.full((1, 1), jnp.sum(blk) / e_real, _f32)
    mn_ref[...] = jnp.full((1, 1), jnp.min(blk), _f32)
    mx_ref[...] = jnp.full((1, 1), jnp.max(blk), _f32)


def _ea_stats(ea):
    e_real = ea.shape[0]
    rows = e_real // 128
    ea2 = ea.reshape(rows, 128)
    out = [jax.ShapeDtypeStruct((1, 1), _f32)] * 3
    return pl.pallas_call(
        functools.partial(_ea_stats_body, e_real=float(e_real)),
        out_shape=out,
    )(ea2)


def _prep_body(h_ref, wl_ref, bl_ref, wr_ref, br_ref, we_ref, att_ref,
               eamean_ref, eamin_ref, eamax_ref,
               xl_ref, xr_ref, self8_ref, usc_ref, utc_ref,
               mnl_ref, mxl_ref, mnr_ref, mxr_ref, *, bn):
    i = pl.program_id(0)
    hb = h_ref[...]
    xlb = jnp.dot(hb, wl_ref[...], preferred_element_type=_f32) + bl_ref[...]
    xrb = jnp.dot(hb, wr_ref[...], preferred_element_type=_f32) + br_ref[...]
    xl_ref[...] = xlb
    xr_ref[...] = xrb
    we = we_ref[...]
    att = att_ref[...]
    mean = eamean_ref[0, 0]
    m = xlb + xrb + mean * we
    m = jnp.maximum(m, 0.2 * m)
    t = (m * att).reshape(bn, _H, _C)
    self8_ref[...] = jnp.sum(t, axis=-1)

    bmnl = jnp.min(xlb, axis=0, keepdims=True)
    bmxl = jnp.max(xlb, axis=0, keepdims=True)
    bmnr = jnp.min(xrb, axis=0, keepdims=True)
    bmxr = jnp.max(xrb, axis=0, keepdims=True)

    @pl.when(i == 0)
    def _():
        mnl_ref[...] = bmnl
        mxl_ref[...] = bmxl
        mnr_ref[...] = bmnr
        mxr_ref[...] = bmxr

    @pl.when(i > 0)
    def _():
        mnl_ref[...] = jnp.minimum(mnl_ref[...], bmnl)
        mxl_ref[...] = jnp.maximum(mxl_ref[...], bmxl)
        mnr_ref[...] = jnp.minimum(mnr_ref[...], bmnr)
        mxr_ref[...] = jnp.maximum(mxr_ref[...], bmxr)

    @pl.when(i == pl.num_programs(0) - 1)
    def _():
        ea_lo = eamin_ref[0, 0]
        ea_hi = eamax_ref[0, 0]
        e_hi = jnp.maximum(we * ea_lo, we * ea_hi)
        e_lo = jnp.minimum(we * ea_lo, we * ea_hi)
        bhi = mxl_ref[...] + mxr_ref[...] + e_hi
        blo = mnl_ref[...] + mnr_ref[...] + e_lo
        mhi = jnp.maximum(bhi, 0.2 * bhi)
        mlo = jnp.maximum(blo, 0.2 * blo)
        chi = jnp.maximum(att * mhi, att * mlo)
        u8 = jnp.sum(chi.reshape(1, _H, _C), axis=-1)
        utc_ref[...] = u8
        usc_ref[...] = jnp.broadcast_to(u8.reshape(_H, 1), (_H, 16))


def _prep(h, p, stats):
    n, din = h.shape
    bn = _pick(n, (2000, 1000, 400, 200, 100, 50, 25, 16, 8, 4, 2, 1))
    g = n // bn
    full = lambda a: pl.BlockSpec(a.shape, lambda i: tuple([0] * a.ndim))
    wl = p['Wl']
    bl = p['bl'].reshape(1, _HC)
    wr = p['Wr']
    br = p['br'].reshape(1, _HC)
    we = p['We'].reshape(1, _HC)
    att = p['att'].reshape(1, _HC)
    eamean, eamin, eamax = stats
    outs = [
        jax.ShapeDtypeStruct((n, _HC), _f32),
        jax.ShapeDtypeStruct((n, _HC), _f32),
        jax.ShapeDtypeStruct((n, _H), _f32),
        jax.ShapeDtypeStruct((_H, 16), _f32),
        jax.ShapeDtypeStruct((1, _H), _f32),
    ]
    return pl.pallas_call(
        functools.partial(_prep_body, bn=bn),
        grid=(g,),
        in_specs=[
            pl.BlockSpec((bn, din), lambda i: (i, 0)),
            full(wl), full(bl), full(wr), full(br), full(we), full(att),
            full(eamean), full(eamin), full(eamax),
        ],
        out_specs=[
            pl.BlockSpec((bn, _HC), lambda i: (i, 0)),
            pl.BlockSpec((bn, _HC), lambda i: (i, 0)),
            pl.BlockSpec((bn, _H), lambda i: (i, 0)),
            pl.BlockSpec((_H, 16), lambda i: (0, 0)),
            pl.BlockSpec((1, _H), lambda i: (0, 0)),
        ],
        out_shape=outs,
        scratch_shapes=[pltpu.VMEM((1, _HC), _f32)] * 4,
    )(h, wl, bl, wr, br, we, att, eamean, eamin, eamax)


def _combine_body(acc_ref, den_ref, xl_ref, self8_ref, utc_ref, bias_ref,
                  out_ref, *, bn, concat):
    accb = acc_ref[...]
    denb = den_ref[...]
    acc64 = jnp.concatenate([accb[0], accb[1]], axis=1)
    den8 = jnp.concatenate([denb[0][:, :4], denb[1][:, :4]], axis=1)
    ws = jnp.exp(self8_ref[...] - utc_ref[...])
    xlb = xl_ref[...]
    wsr = jnp.broadcast_to(ws.reshape(bn, _H, 1), (bn, _H, _C)).reshape(bn, _HC)
    num = acc64 + wsr * xlb
    d = den8 + ws + 1e-16
    dr = jnp.broadcast_to(d.reshape(bn, _H, 1), (bn, _H, _C)).reshape(bn, _HC)
    o = num / dr
    if concat:
        out_ref[...] = jnp.maximum(o + bias_ref[...], 0.0)
    else:
        s = o[:, 0:_C]
        for hh in range(1, _H):
            s = s + o[:, hh * _C:(hh + 1) * _C]
        o8 = s * (1.0 / _H) + bias_ref[...]
        o8 = jnp.maximum(o8, 0.0)
        out_ref[...] = jnp.concatenate([o8, jnp.zeros((bn, 8), _f32)], axis=1)


def _combine(acc, den, xl, self8, utc, bias, concat):
    n = xl.shape[0]
    bn = _pick(n, (2000, 1000, 400, 200, 100, 50, 25, 16, 8, 4, 2, 1))
    g = n // bn
    bias2 = bias.reshape(1, -1)
    odim = _HC if concat else 16
    return pl.pallas_call(
        functools.partial(_combine_body, bn=bn, concat=concat),
        grid=(g,),
        in_specs=[
            pl.BlockSpec((2, bn, 32), lambda i: (0, i, 0)),
            pl.BlockSpec((2, bn, 8), lambda i: (0, i, 0)),
            pl.BlockSpec((bn, _HC), lambda i: (i, 0)),
            pl.BlockSpec((bn, _H), lambda i: (i, 0)),
            pl.BlockSpec((1, _H), lambda i: (0, 0)),
            pl.BlockSpec(bias2.shape, lambda i: (0, 0)),
        ],
        out_specs=pl.BlockSpec((bn, odim), lambda i: (i, 0)),
        out_shape=jax.ShapeDtypeStruct((n, odim), _f32),
    )(acc, den, xl, self8, utc, bias2)


def _head_body(h_ref, w1_ref, b1_ref, w2_ref, b2_ref, out_ref):
    hb = h_ref[...]
    z = jnp.dot(hb, w1_ref[...], preferred_element_type=_f32) + b1_ref[...]
    z = jnp.maximum(z, 0.0)
    z2 = jnp.dot(z, w2_ref[...], preferred_element_type=_f32) + b2_ref[...]
    m = jnp.max(z2, axis=1, keepdims=True)
    zz = z2 - m
    lse = jnp.log(jnp.sum(jnp.exp(zz), axis=1, keepdims=True))
    out_ref[...] = zz - lse


def _mlp_head(feat, p):
    n = feat.shape[0]
    bn = _pick(n, (2000, 1000, 400, 200, 100, 50, 25, 16, 8, 4, 2, 1))
    g = n // bn
    w1 = jnp.concatenate([p['W1'], jnp.zeros((16 - p['W1'].shape[0], p['W1'].shape[1]), _f32)], axis=0)
    b1 = p['b1'].reshape(1, -1)
    w2 = p['W2']
    b2 = p['b2'].reshape(1, -1)
    full = lambda a: pl.BlockSpec(a.shape, lambda i: tuple([0] * a.ndim))
    return pl.pallas_call(
        _head_body,
        grid=(g,),
        in_specs=[
            pl.BlockSpec((bn, 16), lambda i: (i, 0)),
            full(w1), full(b1), full(w2), full(b2),
        ],
        out_specs=pl.BlockSpec((bn, 2), lambda i: (i, 0)),
        out_shape=jax.ShapeDtypeStruct((n, 2), _f32),
    )(feat, w1, b1, w2, b2)


def _mesh():
    return plsc.VectorSubcoreMesh(
        core_axis_name="c", subcore_axis_name="s",
        num_cores=_NC, num_subcores=_NS)


_SC_PARAMS = pltpu.CompilerParams(
    needs_layout_passes=False, use_tc_tiling_on_sc=False)


def _passA_body(xl_hbm, xr_hbm, ei_hbm, ea_hbm, web_hbm, attb_hbm, ub_hbm,
                w_hbm, webuf, attbuf, ubuf, eib, eab, xlb, xrb, wb, sem,
                *, e_real, nchunk):
    c = lax.axis_index("c")
    s = lax.axis_index("s")
    t = c * _NS + s
    pltpu.sync_copy(web_hbm, webuf)
    pltpu.sync_copy(attb_hbm, attbuf)
    pltpu.sync_copy(ub_hbm, ubuf)
    tile_base = t * (nchunk * _EB)

    def chunk(i, carry):
        base = tile_base + i * _EB
        pltpu.sync_copy(ei_hbm.at[:, pl.ds(base, _EB)], eib)
        pltpu.sync_copy(ea_hbm.at[pl.ds(base, _EB)], eab)
        d1 = pltpu.async_copy(xl_hbm.at[eib.at[0]], xlb, sem)
        d2 = pltpu.async_copy(xr_hbm.at[eib.at[1]], xrb, sem)
        d1.wait()
        d2.wait()

        def block(eb, carry2):
            ids = lax.iota(_i32, 16) + eb * 16
            eav = eab[pl.ds(eb * 16, 16)]
            logits = [jnp.zeros((16,), _f32)] * _H
            for j in range(_HC):
                jv = jnp.full((16,), j, _i32)
                xc = plsc.load_gather(xlb, [ids, jv])
                rc = plsc.load_gather(xrb, [ids, jv])
                mm = xc + rc + eav * webuf[j]
                mm = jnp.maximum(mm, 0.2 * mm)
                logits[j // _C] = logits[j // _C] + mm * attbuf[j]
            gid = base + eb * 16 + lax.iota(_i32, 16)
            msk = gid < e_real
            for h in range(_H):
                wv = jnp.exp(logits[h] - ubuf[h])
                wv = jnp.where(msk, wv, 0.0)
                wb[h, pl.ds(eb * 16, 16)] = wv
            return carry2

        lax.fori_loop(0, _EB // 16, block, 0)
        pltpu.sync_copy(wb, w_hbm.at[:, pl.ds(base, _EB)])
        return carry

    lax.fori_loop(0, nchunk, chunk, 0)


def _passA(xl, xr, ei_pad, ea_pad, web, attb, usc, e_real, epad):
    nchunk = epad // (_NW * _EB)
    f = pl.kernel(
        functools.partial(_passA_body, e_real=e_real, nchunk=nchunk),
        out_type=[jax.ShapeDtypeStruct((_H, epad), _f32)],
        mesh=_mesh(),
        compiler_params=_SC_PARAMS,
        scratch_types=[
            pltpu.VMEM((_HC, 16), _f32),
            pltpu.VMEM((_HC, 16), _f32),
            pltpu.VMEM((_H, 16), _f32),
            pltpu.VMEM((2, _EB), _i32),
            pltpu.VMEM((_EB,), _f32),
            pltpu.VMEM((_EB, _HC), _f32),
            pltpu.VMEM((_EB, _HC), _f32),
            pltpu.VMEM((_H, _EB), _f32),
            pltpu.SemaphoreType.DMA,
        ],
    )
    return f(xl, xr, ei_pad, ea_pad, web, attb, usc)[0]


def _passB_body(xl0_hbm, xl1_hbm, ei_hbm, w_hbm, z32_hbm,
                accout, eib, xlc, wbv, msg, sem, acc_sh, *, rpt, nchunk):
    c = lax.axis_index("c")
    s = lax.axis_index("s")
    r0 = s * rpt
    pltpu.sync_copy(z32_hbm.at[pl.ds(r0, rpt)], acc_sh.at[pl.ds(r0, rpt)])
    plsc.subcore_barrier()
    stripe = nchunk * _EB

    def run(core, xsrc_hbm):
        def chunk(i, carry):
            base = s * stripe + i * _EB
            pltpu.sync_copy(ei_hbm.at[:, pl.ds(base, _EB)], eib)
            pltpu.async_copy(xsrc_hbm.at[eib.at[0]], xlc, sem).wait()
            pltpu.sync_copy(
                w_hbm.at[pl.ds(core * 4, 4), pl.ds(base, _EB)], wbv)

            def block(eb, carry2):
                ids = lax.iota(_i32, 16) + eb * 16
                wv = [plsc.load_gather(wbv, [jnp.full((16,), h, _i32), ids])
                      for h in range(4)]
                for j in range(32):
                    jv = jnp.full((16,), j, _i32)
                    col = plsc.load_gather(xlc, [ids, jv])
                    plsc.store_scatter(msg, [ids, jv], col * wv[j // _C])
                return carry2

            lax.fori_loop(0, _EB // 16, block, 0)
            pltpu.sync_copy(msg, acc_sh.at[eib.at[1]], add=True)
            return carry

        lax.fori_loop(0, nchunk, chunk, 0)
        plsc.subcore_barrier()
        pltpu.sync_copy(acc_sh.at[pl.ds(r0, rpt)],
                        accout.at[core, pl.ds(r0, rpt)])

    @pl.when(c == 0)
    def _():
        run(0, xl0_hbm)

    @pl.when(c == 1)
    def _():
        run(1, xl1_hbm)


def _passB(xl0, xl1, ei_pad, w, n, epad):
    rpt = n // _NS
    nchunk = epad // (_NS * _EB)
    z32 = jnp.zeros((n, 32), _f32)
    f = pl.kernel(
        functools.partial(_passB_body, rpt=rpt, nchunk=nchunk),
        out_type=[jax.ShapeDtypeStruct((_NC, n, 32), _f32)],
        mesh=_mesh(),
        compiler_params=_SC_PARAMS,
        scratch_types=[
            pltpu.VMEM((2, _EB), _i32),
            pltpu.VMEM((_EB, 32), _f32),
            pltpu.VMEM((4, _EB), _f32),
            pltpu.VMEM((_EB, 32), _f32),
            pltpu.SemaphoreType.DMA,
            pltpu.VMEM_SHARED((n, 32), _f32),
        ],
    )
    return f(xl0, xl1, ei_pad, w, z32)[0]


def _passBd_body(ei_hbm, w_hbm, z8_hbm, z8row_hbm, denout,
                 dstb, wbv, denb, acc_den_sh, *, rpt, nchunk):
    c = lax.axis_index("c")
    s = lax.axis_index("s")
    r0 = s * rpt
    pltpu.sync_copy(z8_hbm.at[pl.ds(r0, rpt)], acc_den_sh.at[pl.ds(r0, rpt)])
    pltpu.sync_copy(z8row_hbm, denb)
    plsc.subcore_barrier()
    stripe = nchunk * _EB

    def run(core):
        def chunk(i, carry):
            base = s * stripe + i * _EB
            pltpu.sync_copy(ei_hbm.at[:, pl.ds(base, _EB)], dstb)
            pltpu.sync_copy(
                w_hbm.at[pl.ds(core * 4, 4), pl.ds(base, _EB)], wbv)

            def block(eb, carry2):
                ids = lax.iota(_i32, 16) + eb * 16
                for h in range(4):
                    wv = plsc.load_gather(wbv, [jnp.full((16,), h, _i32), ids])
                    plsc.store_scatter(denb, [ids, jnp.full((16,), h, _i32)],
                                       wv)
                return carry2

            lax.fori_loop(0, _EB // 16, block, 0)
            pltpu.sync_copy(denb, acc_den_sh.at[dstb.at[1]], add=True)
            return carry

        lax.fori_loop(0, nchunk, chunk, 0)
        plsc.subcore_barrier()
        pltpu.sync_copy(acc_den_sh.at[pl.ds(r0, rpt)],
                        denout.at[core, pl.ds(r0, rpt)])

    @pl.when(c == 0)
    def _():
        run(0)

    @pl.when(c == 1)
    def _():
        run(1)


def _passBd(ei_pad, w, n, epad):
    rpt = n // _NS
    nchunk = epad // (_NS * _EB)
    z8 = jnp.zeros((n, 8), _f32)
    z8row = jnp.zeros((_EB, 8), _f32)
    f = pl.kernel(
        functools.partial(_passBd_body, rpt=rpt, nchunk=nchunk),
        out_type=[jax.ShapeDtypeStruct((_NC, n, 8), _f32)],
        mesh=_mesh(),
        compiler_params=_SC_PARAMS,
        scratch_types=[
            pltpu.VMEM((2, _EB), _i32),
            pltpu.VMEM((4, _EB), _f32),
            pltpu.VMEM((_EB, 8), _f32),
            pltpu.VMEM_SHARED((n, 8), _f32),
        ],
    )
    return f(ei_pad, w, z8, z8row)[0]


def _passC_body(h16_hbm, ei_hbm, efeat_hbm, eib, g1, g2, efb, sem, *, nchunk):
    c = lax.axis_index("c")
    s = lax.axis_index("s")
    t = c * _NS + s
    tile_base = t * (nchunk * _EB)

    def chunk(i, carry):
        base = tile_base + i * _EB
        pltpu.sync_copy(ei_hbm.at[:, pl.ds(base, _EB)], eib)
        d1 = pltpu.async_copy(h16_hbm.at[eib.at[0]], g1, sem)
        d2 = pltpu.async_copy(h16_hbm.at[eib.at[1]], g2, sem)
        d1.wait()
        d2.wait()

        def row(r, carry2):
            efb[r] = g1[r] + g2[r]
            return carry2

        lax.fori_loop(0, _EB, row, 0)
        pltpu.sync_copy(efb, efeat_hbm.at[pl.ds(base, _EB)])
        return carry

    lax.fori_loop(0, nchunk, chunk, 0)


def _passC(h16, ei_pad, epad):
    nchunk = epad // (_NW * _EB)
    f = pl.kernel(
        functools.partial(_passC_body, nchunk=nchunk),
        out_type=[jax.ShapeDtypeStruct((epad, 16), _f32)],
        mesh=_mesh(),
        compiler_params=_SC_PARAMS,
        scratch_types=[
            pltpu.VMEM((2, _EB), _i32),
            pltpu.VMEM((_EB, 16), _f32),
            pltpu.VMEM((_EB, 16), _f32),
            pltpu.VMEM((_EB, 16), _f32),
            pltpu.SemaphoreType.DMA,
        ],
    )
    return f(h16, ei_pad)[0]


def kernel(x, edge_index, edge_attr, conv_params, node_head_params,
           edge_head_params):
    n = x.shape[0]
    e = edge_index.shape[1]
    unit = _NW * _EB
    epad = ((e + unit - 1) // unit) * unit
    pad = epad - e
    ei_pad = jnp.concatenate(
        [edge_index, jnp.zeros((2, pad), edge_index.dtype)], axis=1)
    ea_flat = edge_attr[:, 0]
    ea_pad = jnp.concatenate([ea_flat, jnp.zeros((pad,), _f32)])

    stats = _ea_stats(ea_flat)

    concat_flags = [True, True, False]
    h = x
    for li in range(3):
        p = conv_params[li]
        xl, xr, self8, usc, utc = _prep(h, p, stats)
        web = jnp.broadcast_to(p['We'].reshape(_HC, 1), (_HC, 16))
        attb = jnp.broadcast_to(p['att'].reshape(_HC, 1), (_HC, 16))
        w = _passA(xl, xr, ei_pad, ea_pad, web, attb, usc, e, epad)
        acc = _passB(xl[:, :32], xl[:, 32:], ei_pad, w, n, epad)
        den = _passBd(ei_pad, w, n, epad)
        h = _combine(acc, den, xl, self8, utc, p['bias'], concat_flags[li])

    node_out = _mlp_head(h, node_head_params)
    efeat = _passC(h, ei_pad, epad)[:e]
    edge_out = _mlp_head(efeat, edge_head_params)
    return (node_out, edge_out)
